# baseline (speedup 1.0000x reference)
import functools

import jax
import jax.numpy as jnp
from jax import lax
from jax.experimental import pallas as pl
from jax.experimental.pallas import tpu as pltpu
from jax.experimental.pallas import tpu_sc as plsc

N = 10000
E = 320000
G = 100
D_IN = 128
D_H = 256
H = 10240
ET = E + N
K = 128
NS = 16
NC = 2
EPAD = ((ET + NS * K - 1) // (NS * K)) * (NS * K)
CHUNK = EPAD // NS
NWIN = CHUNK // K
CHUNK_D = EPAD // (NS * NC)
NWIN_D = CHUNK_D // K
RPT = H // NS
RB = 512
NRB = H // RB
FP = jnp.float32

_mesh = plsc.VectorSubcoreMesh(core_axis_name="c", subcore_axis_name="s")


def _zero_vmem_2d(ref, nrows, ncols):
    z = jnp.zeros((16,), FP)

    def body(i, _):
        for j in range(ncols // 16):
            ref[i, pl.ds(j * 16, 16)] = z
        return 0

    lax.fori_loop(0, nrows, body, 0)


@functools.partial(
    pl.kernel,
    mesh=_mesh,
    out_type=jax.ShapeDtypeStruct((NC * H,), FP),
    scratch_types=[
        pltpu.VMEM((NWIN_D, K), jnp.int32),
        pltpu.VMEM((K,), FP),
        pltpu.VMEM((RPT,), FP),
        pltpu.VMEM_SHARED((H,), FP),
        pltpu.SemaphoreType.DMA,
    ],
)
def _deg_kernel(dst_hbm, deg_hbm, didx_all, ones, zbuf, acc, sem):
    c = lax.axis_index("c")
    s = lax.axis_index("s")
    one = jnp.full((16,), 1.0, FP)
    zero = jnp.zeros((16,), FP)

    def fill(i, _):
        ones[pl.ds(i * 16, 16)] = one
        return 0

    lax.fori_loop(0, K // 16, fill, 0)

    def zfill(i, _):
        zbuf[pl.ds(i * 16, 16)] = zero
        return 0

    lax.fori_loop(0, RPT // 16, zfill, 0)
    pltpu.sync_copy(zbuf, acc.at[pl.ds(s * RPT, RPT)])
    pltpu.sync_copy(dst_hbm.at[c, s], didx_all)
    plsc.subcore_barrier()

    DEPTH = 8

    def body(w, _):
        pltpu.async_copy(ones, acc.at[didx_all.at[w]], sem, add=True)

        @pl.when(w >= DEPTH)
        def _():
            pltpu.make_async_copy(ones, acc.at[didx_all.at[w - DEPTH]],
                                  sem).wait()

        return 0

    lax.fori_loop(0, NWIN_D, body, 0)

    def drain(w, _):
        pltpu.make_async_copy(ones, acc.at[didx_all.at[w]], sem).wait()
        return 0

    lax.fori_loop(NWIN_D - DEPTH, NWIN_D, drain, 0)
    plsc.subcore_barrier()
    pltpu.sync_copy(acc.at[pl.ds(s * RPT, RPT)],
                    deg_hbm.at[pl.ds(c * H + s * RPT, RPT)])


SW = 18
NSUP = NWIN // SW


@functools.partial(
    pl.kernel,
    mesh=_mesh,
    out_type=jax.ShapeDtypeStruct((NC * H, D_H // 2), FP),
    scratch_types=[
        pltpu.VMEM((2, SW, K), jnp.int32),
        pltpu.VMEM((2, SW, K), jnp.int32),
        pltpu.VMEM((2, K, D_H // 2), FP),
        pltpu.VMEM_SHARED((H, D_H // 2), FP),
        pltpu.SemaphoreType.DMA,
        pltpu.SemaphoreType.DMA,
        pltpu.SemaphoreType.DMA,
        pltpu.SemaphoreType.DMA,
        pltpu.SemaphoreType.DMA,
    ],
)
def _agg_kernel(y_hbm, src2_hbm, dst_hbm, out_hbm, sidxS, didxS,
                rows, acc, semG0, semG1, semS0, semS1, semI):
    c = lax.axis_index("c")
    s = lax.axis_index("s")

    _zero_vmem_2d(rows.at[0], K, D_H // 2)
    for b in range(RPT // K):
        pltpu.sync_copy(rows.at[0], acc.at[pl.ds(s * RPT + b * K, K)])
    pltpu.sync_copy(src2_hbm.at[c, s, 0], sidxS.at[0])
    pltpu.sync_copy(dst_hbm.at[s, 0], didxS.at[0])
    plsc.subcore_barrier()

    def g_start(bi, wi, r, sem):
        pltpu.async_copy(y_hbm.at[sidxS.at[bi, wi]], rows.at[r], sem)

    def g_wait(bi, wi, r, sem):
        pltpu.make_async_copy(y_hbm.at[sidxS.at[bi, wi]], rows.at[r],
                              sem).wait()

    def s_start(bi, wi, r, sem):
        pltpu.async_copy(rows.at[r], acc.at[didxS.at[bi, wi]], sem, add=True)

    def s_wait(bi, wi, r, sem):
        pltpu.make_async_copy(rows.at[r], acc.at[didxS.at[bi, wi]],
                              sem).wait()

    g_start(0, 0, 0, semG0)
    g_start(0, 1, 1, semG1)

    def outer(sup, _):
        b = sup % 2

        @pl.when(sup < NSUP - 1)
        def _():
            pltpu.async_copy(src2_hbm.at[c, s, sup + 1], sidxS.at[1 - b], semI)
            pltpu.async_copy(dst_hbm.at[s, sup + 1], didxS.at[1 - b], semI)

        def inner(p, _):
            w0 = 2 * p
            g_wait(b, w0, 0, semG0)
            s_start(b, w0, 0, semS0)
            g_wait(b, w0 + 1, 1, semG1)
            s_start(b, w0 + 1, 1, semS1)
            s_wait(b, w0, 0, semS0)
            g_start(b, w0 + 2, 0, semG0)
            s_wait(b, w0 + 1, 1, semS1)
            g_start(b, w0 + 3, 1, semG1)
            return 0

        lax.fori_loop(0, SW // 2 - 1, inner, 0)
        wt = SW - 2

        @pl.when(sup < NSUP - 1)
        def _():
            pltpu.make_async_copy(src2_hbm.at[c, s, sup + 1],
                                  sidxS.at[1 - b], semI).wait()
            pltpu.make_async_copy(dst_hbm.at[s, sup + 1],
                                  didxS.at[1 - b], semI).wait()

        g_wait(b, wt, 0, semG0)
        s_start(b, wt, 0, semS0)
        g_wait(b, wt + 1, 1, semG1)
        s_start(b, wt + 1, 1, semS1)
        s_wait(b, wt, 0, semS0)

        @pl.when(sup < NSUP - 1)
        def _():
            g_start(1 - b, 0, 0, semG0)

        s_wait(b, wt + 1, 1, semS1)

        @pl.when(sup < NSUP - 1)
        def _():
            g_start(1 - b, 1, 1, semG1)

        return 0

    lax.fori_loop(0, NSUP, outer, 0)
    plsc.subcore_barrier()
    pltpu.sync_copy(acc.at[pl.ds(s * RPT, RPT)],
                    out_hbm.at[pl.ds(c * H + s * RPT, RPT)])


def _dinv(deg_lo, deg_hi):
    d = deg_lo + deg_hi
    return jnp.where(d > 0, lax.rsqrt(d), 0.0)


def _dot(a, b):
    return lax.dot_general(a, b, (((1,), (0,)), ((), ())),
                           precision=lax.Precision.DEFAULT,
                           preferred_element_type=FP)


def _dot0(a, b):
    return lax.dot_general(a, b, (((0,), (0,)), ((), ())),
                           precision=lax.Precision.HIGHEST,
                           preferred_element_type=FP)


def _tc1_body(x_ref, w_ref, dlo_ref, dhi_ref, out_ref):
    di = _dinv(dlo_ref[...], dhi_ref[...])
    y = _dot(x_ref[...], w_ref[...]) * di[:, None]
    out_ref[0] = y[:, : D_H // 2]
    out_ref[1] = y[:, D_H // 2:]


def _tc_mid_body(alo_ref, ahi_ref, w_ref, b_ref, dlo_ref, dhi_ref, out_ref):
    di = _dinv(dlo_ref[...], dhi_ref[...])
    hlo = jnp.maximum(alo_ref[0] * di[:, None] + b_ref[0, : D_H // 2], 0.0)
    hhi = jnp.maximum(ahi_ref[0] * di[:, None] + b_ref[0, D_H // 2:], 0.0)
    y = _dot(jnp.concatenate([hlo, hhi], axis=1), w_ref[...]) * di[:, None]
    out_ref[0] = y[:, : D_H // 2]
    out_ref[1] = y[:, D_H // 2:]


def _tc_final_body(alo_ref, ahi_ref, b3_ref, dlo_ref, dhi_ref, bidx_ref,
                   l1w_ref, l1b_ref, l2w_ref, l2b_ref, out_ref,
                   sums_ref, cnts_ref):
    i = pl.program_id(0)

    @pl.when(i == 0)
    def _():
        sums_ref[...] = jnp.zeros_like(sums_ref)
        cnts_ref[...] = jnp.zeros_like(cnts_ref)

    di = _dinv(dlo_ref[...], dhi_ref[...])
    hlo = jnp.maximum(alo_ref[0] * di[:, None] + b3_ref[0, : D_H // 2], 0.0)
    hhi = jnp.maximum(ahi_ref[0] * di[:, None] + b3_ref[0, D_H // 2:], 0.0)
    seg = bidx_ref[...]
    onehot = (seg[:, None] == lax.broadcasted_iota(jnp.int32, (RB, 128), 1)
              ).astype(FP)
    ct = _dot0(onehot, jnp.ones((RB, 8), FP))
    slo = _dot0(onehot, hlo)
    shi = _dot0(onehot, hhi)
    sums_ref[...] += jnp.concatenate([slo, shi], axis=1)
    cnts_ref[...] += ct

    @pl.when(i == NRB - 1)
    def _():
        pooled = sums_ref[...] / jnp.maximum(cnts_ref[:, 0:1], 1.0)
        g = jnp.maximum(_dot(pooled, l1w_ref[...]) + l1b_ref[...], 0.0)
        out_ref[...] = _dot(g, l2w_ref[...]) + l2b_ref[...]


def _row_spec(block_cols):
    return pl.BlockSpec((RB, block_cols), lambda i: (i, 0))


def _full_spec(shape):
    return pl.BlockSpec(shape, lambda i: tuple(0 for _ in shape))


_vec_spec = pl.BlockSpec((RB,), lambda i: (i,))
_half_lo = pl.BlockSpec((1, RB, D_H // 2), lambda i: (0, i, 0))
_half_hi = pl.BlockSpec((1, RB, D_H // 2), lambda i: (1, i, 0))
_out_split = pl.BlockSpec((2, RB, D_H // 2), lambda i: (0, i, 0))


def _tc1(x_pad, W1, deg_lo, deg_hi):
    return pl.pallas_call(
        _tc1_body,
        grid=(NRB,),
        in_specs=[_row_spec(D_IN), _full_spec((D_IN, D_H)), _vec_spec, _vec_spec],
        out_specs=_out_split,
        out_shape=jax.ShapeDtypeStruct((2, H, D_H // 2), FP),
    )(x_pad, W1, deg_lo, deg_hi)


def _tc_mid(agg, W, b, deg_lo, deg_hi):
    return pl.pallas_call(
        _tc_mid_body,
        grid=(NRB,),
        in_specs=[_half_lo, _half_hi, _full_spec((D_H, D_H)),
                  _full_spec((1, D_H)), _vec_spec, _vec_spec],
        out_specs=_out_split,
        out_shape=jax.ShapeDtypeStruct((2, H, D_H // 2), FP),
    )(agg, agg, W, b, deg_lo, deg_hi)


def _tc_final(agg, b3, deg_lo, deg_hi, bidx, l1w, l1b, l2w_p, l2b_p):
    return pl.pallas_call(
        _tc_final_body,
        grid=(NRB,),
        in_specs=[_half_lo, _half_hi, _full_spec((1, D_H)), _vec_spec, _vec_spec,
                  _vec_spec, _full_spec((D_H, D_H)), _full_spec((1, D_H)),
                  _full_spec((D_H, 128)), _full_spec((1, 128))],
        out_specs=_full_spec((128, 128)),
        out_shape=jax.ShapeDtypeStruct((128, 128), FP),
        scratch_shapes=[pltpu.VMEM((128, D_H), FP), pltpu.VMEM((128, 8), FP)],
    )(agg, agg, b3, deg_lo, deg_hi, bidx, l1w, l1b, l2w_p, l2b_p)


def kernel(x, edge_index, batch_idx, W1, b1, W2, b2, W3, b3,
           lin1_W, lin1_b, lin2_W, lin2_b):
    i32 = jnp.int32
    ei = edge_index.astype(i32)
    loop = jnp.arange(N, dtype=i32)
    npad = EPAD - ET
    padrow = N + (jnp.arange(npad, dtype=i32) % 128)
    src = jnp.concatenate([ei[0], loop, padrow])
    dst = jnp.concatenate([ei[1], loop, padrow])
    src2 = jnp.stack([src, src + H]).reshape(2, NS, NSUP, SW, K)
    dst_w = dst.reshape(NS, NSUP, SW, K)

    deg_flat = _deg_kernel(dst.reshape(NC, NS, NWIN_D, K))
    deg_lo, deg_hi = deg_flat[:H], deg_flat[H:]

    x_pad = jnp.pad(x.astype(FP), ((0, H - N), (0, 0)))
    b1r, b2r, b3r = b1[None, :], b2[None, :], b3[None, :]
    l1b = lin1_b[None, :]
    l2w_p = jnp.pad(lin2_W, ((0, 0), (0, 127)))
    l2b_p = jnp.pad(lin2_b, (0, 127))[None, :]
    bidx = jnp.concatenate(
        [batch_idx.astype(i32),
         G + (jnp.arange(H - N, dtype=i32) % (128 - G))])

    y = _tc1(x_pad, W1, deg_lo, deg_hi)
    agg = _agg_kernel(y.reshape(2 * H, D_H // 2), src2, dst_w).reshape(2, H, D_H // 2)
    y = _tc_mid(agg, W2, b1r, deg_lo, deg_hi)
    agg = _agg_kernel(y.reshape(2 * H, D_H // 2), src2, dst_w).reshape(2, H, D_H // 2)
    y = _tc_mid(agg, W3, b2r, deg_lo, deg_hi)
    agg = _agg_kernel(y.reshape(2 * H, D_H // 2), src2, dst_w).reshape(2, H, D_H // 2)
    out = _tc_final(agg, b3r, deg_lo, deg_hi, bidx, lin1_W, l1b, l2w_p, l2b_p)
    return out[:G, :1]


# speedup vs baseline: 1.3059x; 1.3059x over previous
import functools

import jax
import jax.numpy as jnp
from jax import lax
from jax.experimental import pallas as pl
from jax.experimental.pallas import tpu as pltpu
from jax.experimental.pallas import tpu_sc as plsc

N = 10000
E = 320000
G = 100
D_IN = 128
D_H = 256
H = 10240
ET = E + N
K = 128
NS = 16
NC = 2
EPAD = ((ET + NS * K - 1) // (NS * K)) * (NS * K)
CHUNK = EPAD // NS
NWIN = CHUNK // K
CHUNK_D = EPAD // (NS * NC)
NWIN_D = CHUNK_D // K
RPT = H // NS
RB = 512
NRB = H // RB
FP = jnp.float32

_mesh = plsc.VectorSubcoreMesh(core_axis_name="c", subcore_axis_name="s")


def _zero_vmem_2d(ref, nrows, ncols):
    z = jnp.zeros((16,), FP)

    def body(i, _):
        for j in range(ncols // 16):
            ref[i, pl.ds(j * 16, 16)] = z
        return 0

    lax.fori_loop(0, nrows, body, 0)


@functools.partial(
    pl.kernel,
    mesh=_mesh,
    out_type=jax.ShapeDtypeStruct((NC * H,), FP),
    scratch_types=[
        pltpu.VMEM((NWIN_D, K), jnp.int32),
        pltpu.VMEM((K,), FP),
        pltpu.VMEM((RPT,), FP),
        pltpu.VMEM_SHARED((H,), FP),
        pltpu.SemaphoreType.DMA,
    ],
)
def _deg_kernel(dst_hbm, deg_hbm, didx_all, ones, zbuf, acc, sem):
    c = lax.axis_index("c")
    s = lax.axis_index("s")
    one = jnp.full((16,), 1.0, FP)
    zero = jnp.zeros((16,), FP)

    def fill(i, _):
        ones[pl.ds(i * 16, 16)] = one
        return 0

    lax.fori_loop(0, K // 16, fill, 0)

    def zfill(i, _):
        zbuf[pl.ds(i * 16, 16)] = zero
        return 0

    lax.fori_loop(0, RPT // 16, zfill, 0)
    pltpu.sync_copy(zbuf, acc.at[pl.ds(s * RPT, RPT)])
    pltpu.sync_copy(dst_hbm.at[c, s], didx_all)
    plsc.subcore_barrier()

    DEPTH = 8

    def body(w, _):
        pltpu.async_copy(ones, acc.at[didx_all.at[w]], sem, add=True)

        @pl.when(w >= DEPTH)
        def _():
            pltpu.make_async_copy(ones, acc.at[didx_all.at[w - DEPTH]],
                                  sem).wait()

        return 0

    lax.fori_loop(0, NWIN_D, body, 0)

    def drain(w, _):
        pltpu.make_async_copy(ones, acc.at[didx_all.at[w]], sem).wait()
        return 0

    lax.fori_loop(NWIN_D - DEPTH, NWIN_D, drain, 0)
    plsc.subcore_barrier()
    pltpu.sync_copy(acc.at[pl.ds(s * RPT, RPT)],
                    deg_hbm.at[pl.ds(c * H + s * RPT, RPT)])


SW = 18
NSUP = NWIN // SW


@functools.partial(
    pl.kernel,
    mesh=_mesh,
    out_type=jax.ShapeDtypeStruct((NC * H, D_H // 2), FP),
    scratch_types=[
        pltpu.VMEM((2, SW, K), jnp.int32),
        pltpu.VMEM((2, SW, K), jnp.int32),
        pltpu.VMEM((2, K, D_H // 2), FP),
        pltpu.VMEM_SHARED((H, D_H // 2), FP),
        pltpu.SemaphoreType.DMA,
        pltpu.SemaphoreType.DMA,
        pltpu.SemaphoreType.DMA,
    ],
)
def _agg_kernel(y_hbm, src2_hbm, dst_hbm, out_hbm, sidxS, didxS,
                rows, acc, semG0, semG1, semI):
    c = lax.axis_index("c")
    s = lax.axis_index("s")

    _zero_vmem_2d(rows.at[0], K, D_H // 2)
    for b in range(RPT // K):
        pltpu.sync_copy(rows.at[0], acc.at[pl.ds(s * RPT + b * K, K)])
    pltpu.sync_copy(src2_hbm.at[c, s, 0], sidxS.at[0])
    pltpu.sync_copy(dst_hbm.at[s, 0], didxS.at[0])
    plsc.subcore_barrier()

    pltpu.async_copy(y_hbm.at[sidxS.at[0, 0]], rows.at[0], semG0)

    def outer(sup, _):
        b = sup % 2

        @pl.when(sup < NSUP - 1)
        def _():
            pltpu.async_copy(src2_hbm.at[c, s, sup + 1], sidxS.at[1 - b], semI)
            pltpu.async_copy(dst_hbm.at[s, sup + 1], didxS.at[1 - b], semI)

        def inner(p, _):
            w0 = 2 * p
            pltpu.async_copy(y_hbm.at[sidxS.at[b, w0 + 1]], rows.at[1], semG1)
            pltpu.make_async_copy(y_hbm.at[sidxS.at[b, w0]],
                                  rows.at[0], semG0).wait()
            pltpu.sync_copy(rows.at[0], acc.at[didxS.at[b, w0]], add=True)
            pltpu.async_copy(y_hbm.at[sidxS.at[b, w0 + 2]], rows.at[0], semG0)
            pltpu.make_async_copy(y_hbm.at[sidxS.at[b, w0 + 1]],
                                  rows.at[1], semG1).wait()
            pltpu.sync_copy(rows.at[1], acc.at[didxS.at[b, w0 + 1]], add=True)
            return 0

        lax.fori_loop(0, SW // 2 - 1, inner, 0)
        wt = SW - 2

        @pl.when(sup < NSUP - 1)
        def _():
            pltpu.make_async_copy(src2_hbm.at[c, s, sup + 1],
                                  sidxS.at[1 - b], semI).wait()
            pltpu.make_async_copy(dst_hbm.at[s, sup + 1],
                                  didxS.at[1 - b], semI).wait()

        pltpu.async_copy(y_hbm.at[sidxS.at[b, wt + 1]], rows.at[1], semG1)
        pltpu.make_async_copy(y_hbm.at[sidxS.at[b, wt]],
                              rows.at[0], semG0).wait()
        pltpu.sync_copy(rows.at[0], acc.at[didxS.at[b, wt]], add=True)

        @pl.when(sup < NSUP - 1)
        def _():
            pltpu.async_copy(y_hbm.at[sidxS.at[1 - b, 0]], rows.at[0], semG0)

        pltpu.make_async_copy(y_hbm.at[sidxS.at[b, wt + 1]],
                              rows.at[1], semG1).wait()
        pltpu.sync_copy(rows.at[1], acc.at[didxS.at[b, wt + 1]], add=True)
        return 0

    lax.fori_loop(0, NSUP, outer, 0)
    plsc.subcore_barrier()
    pltpu.sync_copy(acc.at[pl.ds(s * RPT, RPT)],
                    out_hbm.at[pl.ds(c * H + s * RPT, RPT)])


def _dinv(deg_lo, deg_hi):
    d = deg_lo + deg_hi
    return jnp.where(d > 0, lax.rsqrt(d), 0.0)


def _dot(a, b):
    return lax.dot_general(a, b, (((1,), (0,)), ((), ())),
                           precision=lax.Precision.DEFAULT,
                           preferred_element_type=FP)


def _dot0(a, b):
    return lax.dot_general(a, b, (((0,), (0,)), ((), ())),
                           precision=lax.Precision.HIGHEST,
                           preferred_element_type=FP)


def _tc1_body(x_ref, w_ref, dlo_ref, dhi_ref, out_ref):
    di = _dinv(dlo_ref[...], dhi_ref[...])
    y = _dot(x_ref[...], w_ref[...]) * di[:, None]
    out_ref[0] = y[:, : D_H // 2]
    out_ref[1] = y[:, D_H // 2:]


def _tc_mid_body(a_ref, w_ref, b_ref, dlo_ref, dhi_ref, out_ref):
    di = _dinv(dlo_ref[...], dhi_ref[...])
    hlo = jnp.maximum(a_ref[0] * di[:, None] + b_ref[0, : D_H // 2], 0.0)
    hhi = jnp.maximum(a_ref[1] * di[:, None] + b_ref[0, D_H // 2:], 0.0)
    y = _dot(jnp.concatenate([hlo, hhi], axis=1), w_ref[...]) * di[:, None]
    out_ref[0] = y[:, : D_H // 2]
    out_ref[1] = y[:, D_H // 2:]


def _tc_final_body(a_ref, b3_ref, dlo_ref, dhi_ref, bidx_ref,
                   l1w_ref, l1b_ref, l2w_ref, l2b_ref, out_ref,
                   sums_ref, cnts_ref):
    i = pl.program_id(0)

    @pl.when(i == 0)
    def _():
        sums_ref[...] = jnp.zeros_like(sums_ref)
        cnts_ref[...] = jnp.zeros_like(cnts_ref)

    di = _dinv(dlo_ref[...], dhi_ref[...])
    hlo = jnp.maximum(a_ref[0] * di[:, None] + b3_ref[0, : D_H // 2], 0.0)
    hhi = jnp.maximum(a_ref[1] * di[:, None] + b3_ref[0, D_H // 2:], 0.0)
    seg = bidx_ref[...]
    onehot = (seg[:, None] == lax.broadcasted_iota(jnp.int32, (RB, 128), 1)
              ).astype(FP)
    ct = _dot0(onehot, jnp.ones((RB, 8), FP))
    slo = _dot0(onehot, hlo)
    shi = _dot0(onehot, hhi)
    sums_ref[...] += jnp.concatenate([slo, shi], axis=1)
    cnts_ref[...] += ct

    @pl.when(i == NRB - 1)
    def _():
        pooled = sums_ref[...] / jnp.maximum(cnts_ref[:, 0:1], 1.0)
        g = jnp.maximum(_dot(pooled, l1w_ref[...]) + l1b_ref[...], 0.0)
        out_ref[...] = _dot(g, l2w_ref[...]) + l2b_ref[...]


def _row_spec(block_cols):
    return pl.BlockSpec((RB, block_cols), lambda i: (i, 0))


def _full_spec(shape):
    return pl.BlockSpec(shape, lambda i: tuple(0 for _ in shape))


_vec_spec = pl.BlockSpec((RB,), lambda i: (i,))
_both_halves = pl.BlockSpec((2, RB, D_H // 2), lambda i: (0, i, 0))
_out_split = pl.BlockSpec((2, RB, D_H // 2), lambda i: (0, i, 0))


def _tc1(x_pad, W1, deg_lo, deg_hi):
    return pl.pallas_call(
        _tc1_body,
        grid=(NRB,),
        in_specs=[_row_spec(D_IN), _full_spec((D_IN, D_H)), _vec_spec, _vec_spec],
        out_specs=_out_split,
        out_shape=jax.ShapeDtypeStruct((2, H, D_H // 2), FP),
    )(x_pad, W1, deg_lo, deg_hi)


def _tc_mid(agg, W, b, deg_lo, deg_hi):
    return pl.pallas_call(
        _tc_mid_body,
        grid=(NRB,),
        in_specs=[_both_halves, _full_spec((D_H, D_H)),
                  _full_spec((1, D_H)), _vec_spec, _vec_spec],
        out_specs=_out_split,
        out_shape=jax.ShapeDtypeStruct((2, H, D_H // 2), FP),
    )(agg, W, b, deg_lo, deg_hi)


def _tc_final(agg, b3, deg_lo, deg_hi, bidx, l1w, l1b, l2w_p, l2b_p):
    return pl.pallas_call(
        _tc_final_body,
        grid=(NRB,),
        in_specs=[_both_halves, _full_spec((1, D_H)), _vec_spec, _vec_spec,
                  _vec_spec, _full_spec((D_H, D_H)), _full_spec((1, D_H)),
                  _full_spec((D_H, 128)), _full_spec((1, 128))],
        out_specs=_full_spec((128, 128)),
        out_shape=jax.ShapeDtypeStruct((128, 128), FP),
        scratch_shapes=[pltpu.VMEM((128, D_H), FP), pltpu.VMEM((128, 8), FP)],
    )(agg, b3, deg_lo, deg_hi, bidx, l1w, l1b, l2w_p, l2b_p)


def kernel(x, edge_index, batch_idx, W1, b1, W2, b2, W3, b3,
           lin1_W, lin1_b, lin2_W, lin2_b):
    i32 = jnp.int32
    ei = edge_index.astype(i32)
    loop = jnp.arange(N, dtype=i32)
    npad = EPAD - ET
    padrow = N + (jnp.arange(npad, dtype=i32) % 128)
    src = jnp.concatenate([ei[0], loop, padrow])
    dst = jnp.concatenate([ei[1], loop, padrow])
    src2 = jnp.stack([src, src + H]).reshape(2, NS, NSUP, SW, K)
    dst_w = dst.reshape(NS, NSUP, SW, K)

    deg_flat = _deg_kernel(dst.reshape(NC, NS, NWIN_D, K))
    deg_lo, deg_hi = deg_flat[:H], deg_flat[H:]

    x_pad = jnp.pad(x.astype(FP), ((0, H - N), (0, 0)))
    b1r, b2r, b3r = b1[None, :], b2[None, :], b3[None, :]
    l1b = lin1_b[None, :]
    l2w_p = jnp.pad(lin2_W, ((0, 0), (0, 127)))
    l2b_p = jnp.pad(lin2_b, (0, 127))[None, :]
    bidx = jnp.concatenate(
        [batch_idx.astype(i32),
         G + (jnp.arange(H - N, dtype=i32) % (128 - G))])

    y = _tc1(x_pad, W1, deg_lo, deg_hi)
    agg = _agg_kernel(y.reshape(2 * H, D_H // 2), src2, dst_w).reshape(2, H, D_H // 2)
    y = _tc_mid(agg, W2, b1r, deg_lo, deg_hi)
    agg = _agg_kernel(y.reshape(2 * H, D_H // 2), src2, dst_w).reshape(2, H, D_H // 2)
    y = _tc_mid(agg, W3, b2r, deg_lo, deg_hi)
    agg = _agg_kernel(y.reshape(2 * H, D_H // 2), src2, dst_w).reshape(2, H, D_H // 2)
    out = _tc_final(agg, b3r, deg_lo, deg_hi, bidx, lin1_W, l1b, l2w_p, l2b_p)
    return out[:G, :1]


# speedup vs baseline: 1.3384x; 1.0249x over previous
import functools

import jax
import jax.numpy as jnp
from jax import lax
from jax.experimental import pallas as pl
from jax.experimental.pallas import tpu as pltpu
from jax.experimental.pallas import tpu_sc as plsc

N = 10000
E = 320000
G = 100
D_IN = 128
D_H = 256
H = 10240
ET = E + N
K = 128
NS = 16
NC = 2
EPAD = ((ET + NS * K - 1) // (NS * K)) * (NS * K)
CHUNK = EPAD // NS
NWIN = CHUNK // K
CHUNK_D = EPAD // (NS * NC)
NWIN_D = CHUNK_D // K
RPT = H // NS
RB = 1024
NRB = H // RB
FP = jnp.float32

_mesh = plsc.VectorSubcoreMesh(core_axis_name="c", subcore_axis_name="s")


def _zero_vmem_2d(ref, nrows, ncols):
    z = jnp.zeros((16,), FP)

    def body(i, _):
        for j in range(ncols // 16):
            ref[i, pl.ds(j * 16, 16)] = z
        return 0

    lax.fori_loop(0, nrows, body, 0)


@functools.partial(
    pl.kernel,
    mesh=_mesh,
    out_type=jax.ShapeDtypeStruct((NC * H,), FP),
    scratch_types=[
        pltpu.VMEM((NWIN_D, K), jnp.int32),
        pltpu.VMEM((K,), FP),
        pltpu.VMEM((RPT,), FP),
        pltpu.VMEM_SHARED((H,), FP),
        pltpu.SemaphoreType.DMA,
    ],
)
def _deg_kernel(dst_hbm, deg_hbm, didx_all, ones, zbuf, acc, sem):
    c = lax.axis_index("c")
    s = lax.axis_index("s")
    one = jnp.full((16,), 1.0, FP)
    zero = jnp.zeros((16,), FP)

    def fill(i, _):
        ones[pl.ds(i * 16, 16)] = one
        return 0

    lax.fori_loop(0, K // 16, fill, 0)

    def zfill(i, _):
        zbuf[pl.ds(i * 16, 16)] = zero
        return 0

    lax.fori_loop(0, RPT // 16, zfill, 0)
    pltpu.sync_copy(zbuf, acc.at[pl.ds(s * RPT, RPT)])
    pltpu.sync_copy(dst_hbm.at[c, s], didx_all)
    plsc.subcore_barrier()

    DEPTH = 8

    def body(w, _):
        pltpu.async_copy(ones, acc.at[didx_all.at[w]], sem, add=True)

        @pl.when(w >= DEPTH)
        def _():
            pltpu.make_async_copy(ones, acc.at[didx_all.at[w - DEPTH]],
                                  sem).wait()

        return 0

    lax.fori_loop(0, NWIN_D, body, 0)

    def drain(w, _):
        pltpu.make_async_copy(ones, acc.at[didx_all.at[w]], sem).wait()
        return 0

    lax.fori_loop(NWIN_D - DEPTH, NWIN_D, drain, 0)
    plsc.subcore_barrier()
    pltpu.sync_copy(acc.at[pl.ds(s * RPT, RPT)],
                    deg_hbm.at[pl.ds(c * H + s * RPT, RPT)])


SW = 18
NSUP = NWIN // SW


@functools.partial(
    pl.kernel,
    mesh=_mesh,
    out_type=jax.ShapeDtypeStruct((NC * H, D_H // 2), FP),
    scratch_types=[
        pltpu.VMEM((2, SW, K), jnp.int32),
        pltpu.VMEM((2, SW, K), jnp.int32),
        pltpu.VMEM((2, K, D_H // 2), FP),
        pltpu.VMEM_SHARED((H, D_H // 2), FP),
        pltpu.SemaphoreType.DMA,
        pltpu.SemaphoreType.DMA,
        pltpu.SemaphoreType.DMA,
    ],
)
def _agg_kernel(y_hbm, src2_hbm, dst_hbm, out_hbm, sidxS, didxS,
                rows, acc, semG0, semG1, semI):
    c = lax.axis_index("c")
    s = lax.axis_index("s")

    _zero_vmem_2d(rows.at[0], K, D_H // 2)
    for b in range(RPT // K):
        pltpu.sync_copy(rows.at[0], acc.at[pl.ds(s * RPT + b * K, K)])
    pltpu.sync_copy(src2_hbm.at[c, s, 0], sidxS.at[0])
    pltpu.sync_copy(dst_hbm.at[s, 0], didxS.at[0])
    plsc.subcore_barrier()

    pltpu.async_copy(y_hbm.at[sidxS.at[0, 0]], rows.at[0], semG0)

    def outer(sup, _):
        b = sup % 2

        @pl.when(sup < NSUP - 1)
        def _():
            pltpu.async_copy(src2_hbm.at[c, s, sup + 1], sidxS.at[1 - b], semI)
            pltpu.async_copy(dst_hbm.at[s, sup + 1], didxS.at[1 - b], semI)

        def inner(p, _):
            w0 = 2 * p
            pltpu.async_copy(y_hbm.at[sidxS.at[b, w0 + 1]], rows.at[1], semG1)
            pltpu.make_async_copy(y_hbm.at[sidxS.at[b, w0]],
                                  rows.at[0], semG0).wait()
            pltpu.sync_copy(rows.at[0], acc.at[didxS.at[b, w0]], add=True)
            pltpu.async_copy(y_hbm.at[sidxS.at[b, w0 + 2]], rows.at[0], semG0)
            pltpu.make_async_copy(y_hbm.at[sidxS.at[b, w0 + 1]],
                                  rows.at[1], semG1).wait()
            pltpu.sync_copy(rows.at[1], acc.at[didxS.at[b, w0 + 1]], add=True)
            return 0

        lax.fori_loop(0, SW // 2 - 1, inner, 0)
        wt = SW - 2

        @pl.when(sup < NSUP - 1)
        def _():
            pltpu.make_async_copy(src2_hbm.at[c, s, sup + 1],
                                  sidxS.at[1 - b], semI).wait()
            pltpu.make_async_copy(dst_hbm.at[s, sup + 1],
                                  didxS.at[1 - b], semI).wait()

        pltpu.async_copy(y_hbm.at[sidxS.at[b, wt + 1]], rows.at[1], semG1)
        pltpu.make_async_copy(y_hbm.at[sidxS.at[b, wt]],
                              rows.at[0], semG0).wait()
        pltpu.sync_copy(rows.at[0], acc.at[didxS.at[b, wt]], add=True)

        @pl.when(sup < NSUP - 1)
        def _():
            pltpu.async_copy(y_hbm.at[sidxS.at[1 - b, 0]], rows.at[0], semG0)

        pltpu.make_async_copy(y_hbm.at[sidxS.at[b, wt + 1]],
                              rows.at[1], semG1).wait()
        pltpu.sync_copy(rows.at[1], acc.at[didxS.at[b, wt + 1]], add=True)
        return 0

    lax.fori_loop(0, NSUP, outer, 0)
    plsc.subcore_barrier()
    pltpu.sync_copy(acc.at[pl.ds(s * RPT, RPT)],
                    out_hbm.at[pl.ds(c * H + s * RPT, RPT)])


def _dinv(deg_lo, deg_hi):
    d = deg_lo + deg_hi
    return jnp.where(d > 0, lax.rsqrt(d), 0.0)


def _dot(a, b):
    return lax.dot_general(a, b, (((1,), (0,)), ((), ())),
                           precision=lax.Precision.DEFAULT,
                           preferred_element_type=FP)


def _dot0(a, b):
    return lax.dot_general(a, b, (((0,), (0,)), ((), ())),
                           precision=lax.Precision.HIGHEST,
                           preferred_element_type=FP)


def _tc1_body(x_ref, w_ref, dlo_ref, dhi_ref, out_ref):
    di = _dinv(dlo_ref[...], dhi_ref[...])
    y = _dot(x_ref[...], w_ref[...]) * di[:, None]
    out_ref[0] = y[:, : D_H // 2]
    out_ref[1] = y[:, D_H // 2:]


def _tc_mid_body(a_ref, w_ref, b_ref, dlo_ref, dhi_ref, out_ref):
    di = _dinv(dlo_ref[...], dhi_ref[...])
    hlo = jnp.maximum(a_ref[0] * di[:, None] + b_ref[0, : D_H // 2], 0.0)
    hhi = jnp.maximum(a_ref[1] * di[:, None] + b_ref[0, D_H // 2:], 0.0)
    y = _dot(jnp.concatenate([hlo, hhi], axis=1), w_ref[...]) * di[:, None]
    out_ref[0] = y[:, : D_H // 2]
    out_ref[1] = y[:, D_H // 2:]


def _tc_final_body(a_ref, b3_ref, dlo_ref, dhi_ref, bidx_ref,
                   l1w_ref, l1b_ref, l2w_ref, l2b_ref, out_ref,
                   sums_ref, cnts_ref):
    i = pl.program_id(0)

    @pl.when(i == 0)
    def _():
        sums_ref[...] = jnp.zeros_like(sums_ref)
        cnts_ref[...] = jnp.zeros_like(cnts_ref)

    di = _dinv(dlo_ref[...], dhi_ref[...])
    hlo = jnp.maximum(a_ref[0] * di[:, None] + b3_ref[0, : D_H // 2], 0.0)
    hhi = jnp.maximum(a_ref[1] * di[:, None] + b3_ref[0, D_H // 2:], 0.0)
    seg = bidx_ref[...]
    onehot = (seg[:, None] == lax.broadcasted_iota(jnp.int32, (RB, 128), 1)
              ).astype(FP)
    ct = _dot0(onehot, jnp.ones((RB, 8), FP))
    slo = _dot0(onehot, hlo)
    shi = _dot0(onehot, hhi)
    sums_ref[...] += jnp.concatenate([slo, shi], axis=1)
    cnts_ref[...] += ct

    @pl.when(i == NRB - 1)
    def _():
        pooled = sums_ref[...] / jnp.maximum(cnts_ref[:, 0:1], 1.0)
        g = jnp.maximum(_dot(pooled, l1w_ref[...]) + l1b_ref[...], 0.0)
        out_ref[...] = _dot(g, l2w_ref[...]) + l2b_ref[...]


def _row_spec(block_cols):
    return pl.BlockSpec((RB, block_cols), lambda i: (i, 0))


def _full_spec(shape):
    return pl.BlockSpec(shape, lambda i: tuple(0 for _ in shape))


_vec_spec = pl.BlockSpec((RB,), lambda i: (i,))
_both_halves = pl.BlockSpec((2, RB, D_H // 2), lambda i: (0, i, 0))
_out_split = pl.BlockSpec((2, RB, D_H // 2), lambda i: (0, i, 0))


def _tc1(x_pad, W1, deg_lo, deg_hi):
    return pl.pallas_call(
        _tc1_body,
        grid=(NRB,),
        in_specs=[_row_spec(D_IN), _full_spec((D_IN, D_H)), _vec_spec, _vec_spec],
        out_specs=_out_split,
        out_shape=jax.ShapeDtypeStruct((2, H, D_H // 2), FP),
    )(x_pad, W1, deg_lo, deg_hi)


def _tc_mid(agg, W, b, deg_lo, deg_hi):
    return pl.pallas_call(
        _tc_mid_body,
        grid=(NRB,),
        in_specs=[_both_halves, _full_spec((D_H, D_H)),
                  _full_spec((1, D_H)), _vec_spec, _vec_spec],
        out_specs=_out_split,
        out_shape=jax.ShapeDtypeStruct((2, H, D_H // 2), FP),
    )(agg, W, b, deg_lo, deg_hi)


def _tc_final(agg, b3, deg_lo, deg_hi, bidx, l1w, l1b, l2w_p, l2b_p):
    return pl.pallas_call(
        _tc_final_body,
        grid=(NRB,),
        in_specs=[_both_halves, _full_spec((1, D_H)), _vec_spec, _vec_spec,
                  _vec_spec, _full_spec((D_H, D_H)), _full_spec((1, D_H)),
                  _full_spec((D_H, 128)), _full_spec((1, 128))],
        out_specs=_full_spec((128, 128)),
        out_shape=jax.ShapeDtypeStruct((128, 128), FP),
        scratch_shapes=[pltpu.VMEM((128, D_H), FP), pltpu.VMEM((128, 8), FP)],
    )(agg, b3, deg_lo, deg_hi, bidx, l1w, l1b, l2w_p, l2b_p)


def kernel(x, edge_index, batch_idx, W1, b1, W2, b2, W3, b3,
           lin1_W, lin1_b, lin2_W, lin2_b):
    i32 = jnp.int32
    ei = edge_index.astype(i32)
    loop = jnp.arange(N, dtype=i32)
    npad = EPAD - ET
    padrow = N + (jnp.arange(npad, dtype=i32) % 128)
    src = jnp.concatenate([ei[0], loop, padrow])
    dst = jnp.concatenate([ei[1], loop, padrow])
    src2 = jnp.stack([src, src + H]).reshape(2, NS, NSUP, SW, K)
    dst_w = dst.reshape(NS, NSUP, SW, K)

    deg_flat = _deg_kernel(dst.reshape(NC, NS, NWIN_D, K))
    deg_lo, deg_hi = deg_flat[:H], deg_flat[H:]

    x_pad = jnp.pad(x.astype(FP), ((0, H - N), (0, 0)))
    b1r, b2r, b3r = b1[None, :], b2[None, :], b3[None, :]
    l1b = lin1_b[None, :]
    l2w_p = jnp.pad(lin2_W, ((0, 0), (0, 127)))
    l2b_p = jnp.pad(lin2_b, (0, 127))[None, :]
    bidx = jnp.concatenate(
        [batch_idx.astype(i32),
         G + (jnp.arange(H - N, dtype=i32) % (128 - G))])

    y = _tc1(x_pad, W1, deg_lo, deg_hi)
    agg = _agg_kernel(y.reshape(2 * H, D_H // 2), src2, dst_w).reshape(2, H, D_H // 2)
    y = _tc_mid(agg, W2, b1r, deg_lo, deg_hi)
    agg = _agg_kernel(y.reshape(2 * H, D_H // 2), src2, dst_w).reshape(2, H, D_H // 2)
    y = _tc_mid(agg, W3, b2r, deg_lo, deg_hi)
    agg = _agg_kernel(y.reshape(2 * H, D_H // 2), src2, dst_w).reshape(2, H, D_H // 2)
    out = _tc_final(agg, b3r, deg_lo, deg_hi, bidx, lin1_W, l1b, l2w_p, l2b_p)
    return out[:G, :1]


# speedup vs baseline: 1.3496x; 1.0084x over previous
import functools

import jax
import jax.numpy as jnp
from jax import lax
from jax.experimental import pallas as pl
from jax.experimental.pallas import tpu as pltpu
from jax.experimental.pallas import tpu_sc as plsc

N = 10000
E = 320000
G = 100
D_IN = 128
D_H = 256
H = 10240
ET = E + N
K = 128
NS = 16
NC = 2
EPAD = ((ET + NS * K - 1) // (NS * K)) * (NS * K)
CHUNK = EPAD // NS
NWIN = CHUNK // K
CHUNK_D = EPAD // (NS * NC)
NWIN_D = CHUNK_D // K
RPT = H // NS
RB = 2048
NRB = H // RB
FP = jnp.float32

_mesh = plsc.VectorSubcoreMesh(core_axis_name="c", subcore_axis_name="s")


def _zero_vmem_2d(ref, nrows, ncols):
    z = jnp.zeros((16,), FP)

    def body(i, _):
        for j in range(ncols // 16):
            ref[i, pl.ds(j * 16, 16)] = z
        return 0

    lax.fori_loop(0, nrows, body, 0)


@functools.partial(
    pl.kernel,
    mesh=_mesh,
    out_type=jax.ShapeDtypeStruct((NC * H,), FP),
    scratch_types=[
        pltpu.VMEM((NWIN_D, K), jnp.int32),
        pltpu.VMEM((K,), FP),
        pltpu.VMEM((RPT,), FP),
        pltpu.VMEM_SHARED((H,), FP),
        pltpu.SemaphoreType.DMA,
    ],
)
def _deg_kernel(dst_hbm, deg_hbm, didx_all, ones, zbuf, acc, sem):
    c = lax.axis_index("c")
    s = lax.axis_index("s")
    one = jnp.full((16,), 1.0, FP)
    zero = jnp.zeros((16,), FP)

    def fill(i, _):
        ones[pl.ds(i * 16, 16)] = one
        return 0

    lax.fori_loop(0, K // 16, fill, 0)

    def zfill(i, _):
        zbuf[pl.ds(i * 16, 16)] = zero
        return 0

    lax.fori_loop(0, RPT // 16, zfill, 0)
    pltpu.sync_copy(zbuf, acc.at[pl.ds(s * RPT, RPT)])
    pltpu.sync_copy(dst_hbm.at[c, s], didx_all)
    plsc.subcore_barrier()

    DEPTH = 8

    def body(w, _):
        pltpu.async_copy(ones, acc.at[didx_all.at[w]], sem, add=True)

        @pl.when(w >= DEPTH)
        def _():
            pltpu.make_async_copy(ones, acc.at[didx_all.at[w - DEPTH]],
                                  sem).wait()

        return 0

    lax.fori_loop(0, NWIN_D, body, 0)

    def drain(w, _):
        pltpu.make_async_copy(ones, acc.at[didx_all.at[w]], sem).wait()
        return 0

    lax.fori_loop(NWIN_D - DEPTH, NWIN_D, drain, 0)
    plsc.subcore_barrier()
    pltpu.sync_copy(acc.at[pl.ds(s * RPT, RPT)],
                    deg_hbm.at[pl.ds(c * H + s * RPT, RPT)])


SW = 18
NSUP = NWIN // SW


@functools.partial(
    pl.kernel,
    mesh=_mesh,
    out_type=jax.ShapeDtypeStruct((NC * H, D_H // 2), FP),
    scratch_types=[
        pltpu.VMEM((2, SW, K), jnp.int32),
        pltpu.VMEM((2, SW, K), jnp.int32),
        pltpu.VMEM((2, K, D_H // 2), FP),
        pltpu.VMEM_SHARED((H, D_H // 2), FP),
        pltpu.SemaphoreType.DMA,
        pltpu.SemaphoreType.DMA,
        pltpu.SemaphoreType.DMA,
    ],
)
def _agg_kernel(y_hbm, src2_hbm, dst_hbm, out_hbm, sidxS, didxS,
                rows, acc, semG0, semG1, semI):
    c = lax.axis_index("c")
    s = lax.axis_index("s")

    _zero_vmem_2d(rows.at[0], K, D_H // 2)
    for b in range(RPT // K):
        pltpu.sync_copy(rows.at[0], acc.at[pl.ds(s * RPT + b * K, K)])
    pltpu.sync_copy(src2_hbm.at[c, s, 0], sidxS.at[0])
    pltpu.sync_copy(dst_hbm.at[s, 0], didxS.at[0])
    plsc.subcore_barrier()

    pltpu.async_copy(y_hbm.at[sidxS.at[0, 0]], rows.at[0], semG0)

    def outer(sup, _):
        b = sup % 2

        @pl.when(sup < NSUP - 1)
        def _():
            pltpu.async_copy(src2_hbm.at[c, s, sup + 1], sidxS.at[1 - b], semI)
            pltpu.async_copy(dst_hbm.at[s, sup + 1], didxS.at[1 - b], semI)

        def inner(p, _):
            w0 = 2 * p
            pltpu.async_copy(y_hbm.at[sidxS.at[b, w0 + 1]], rows.at[1], semG1)
            pltpu.make_async_copy(y_hbm.at[sidxS.at[b, w0]],
                                  rows.at[0], semG0).wait()
            pltpu.sync_copy(rows.at[0], acc.at[didxS.at[b, w0]], add=True)
            pltpu.async_copy(y_hbm.at[sidxS.at[b, w0 + 2]], rows.at[0], semG0)
            pltpu.make_async_copy(y_hbm.at[sidxS.at[b, w0 + 1]],
                                  rows.at[1], semG1).wait()
            pltpu.sync_copy(rows.at[1], acc.at[didxS.at[b, w0 + 1]], add=True)
            return 0

        lax.fori_loop(0, SW // 2 - 1, inner, 0)
        wt = SW - 2

        @pl.when(sup < NSUP - 1)
        def _():
            pltpu.make_async_copy(src2_hbm.at[c, s, sup + 1],
                                  sidxS.at[1 - b], semI).wait()
            pltpu.make_async_copy(dst_hbm.at[s, sup + 1],
                                  didxS.at[1 - b], semI).wait()

        pltpu.async_copy(y_hbm.at[sidxS.at[b, wt + 1]], rows.at[1], semG1)
        pltpu.make_async_copy(y_hbm.at[sidxS.at[b, wt]],
                              rows.at[0], semG0).wait()
        pltpu.sync_copy(rows.at[0], acc.at[didxS.at[b, wt]], add=True)

        @pl.when(sup < NSUP - 1)
        def _():
            pltpu.async_copy(y_hbm.at[sidxS.at[1 - b, 0]], rows.at[0], semG0)

        pltpu.make_async_copy(y_hbm.at[sidxS.at[b, wt + 1]],
                              rows.at[1], semG1).wait()
        pltpu.sync_copy(rows.at[1], acc.at[didxS.at[b, wt + 1]], add=True)
        return 0

    lax.fori_loop(0, NSUP, outer, 0)
    plsc.subcore_barrier()
    pltpu.sync_copy(acc.at[pl.ds(s * RPT, RPT)],
                    out_hbm.at[pl.ds(c * H + s * RPT, RPT)])


def _dinv(deg_lo, deg_hi):
    d = deg_lo + deg_hi
    return jnp.where(d > 0, lax.rsqrt(d), 0.0)


def _dot(a, b):
    return lax.dot_general(a, b, (((1,), (0,)), ((), ())),
                           precision=lax.Precision.DEFAULT,
                           preferred_element_type=FP)


def _dot0(a, b):
    return lax.dot_general(a, b, (((0,), (0,)), ((), ())),
                           precision=lax.Precision.HIGHEST,
                           preferred_element_type=FP)


def _tc1_body(x_ref, w_ref, dlo_ref, dhi_ref, out_ref):
    di = _dinv(dlo_ref[...], dhi_ref[...])
    y = _dot(x_ref[...], w_ref[...]) * di[:, None]
    out_ref[0] = y[:, : D_H // 2]
    out_ref[1] = y[:, D_H // 2:]


def _tc_mid_body(a_ref, w_ref, b_ref, dlo_ref, dhi_ref, out_ref):
    di = _dinv(dlo_ref[...], dhi_ref[...])
    hlo = jnp.maximum(a_ref[0] * di[:, None] + b_ref[0, : D_H // 2], 0.0)
    hhi = jnp.maximum(a_ref[1] * di[:, None] + b_ref[0, D_H // 2:], 0.0)
    y = _dot(jnp.concatenate([hlo, hhi], axis=1), w_ref[...]) * di[:, None]
    out_ref[0] = y[:, : D_H // 2]
    out_ref[1] = y[:, D_H // 2:]


def _tc_final_body(a_ref, b3_ref, dlo_ref, dhi_ref, bidx_ref,
                   l1w_ref, l1b_ref, l2w_ref, l2b_ref, out_ref,
                   sums_ref, cnts_ref):
    i = pl.program_id(0)

    @pl.when(i == 0)
    def _():
        sums_ref[...] = jnp.zeros_like(sums_ref)
        cnts_ref[...] = jnp.zeros_like(cnts_ref)

    di = _dinv(dlo_ref[...], dhi_ref[...])
    hlo = jnp.maximum(a_ref[0] * di[:, None] + b3_ref[0, : D_H // 2], 0.0)
    hhi = jnp.maximum(a_ref[1] * di[:, None] + b3_ref[0, D_H // 2:], 0.0)
    seg = bidx_ref[...]
    onehot = (seg[:, None] == lax.broadcasted_iota(jnp.int32, (RB, 128), 1)
              ).astype(FP)
    ct = _dot0(onehot, jnp.ones((RB, 8), FP))
    slo = _dot0(onehot, hlo)
    shi = _dot0(onehot, hhi)
    sums_ref[...] += jnp.concatenate([slo, shi], axis=1)
    cnts_ref[...] += ct

    @pl.when(i == NRB - 1)
    def _():
        pooled = sums_ref[...] / jnp.maximum(cnts_ref[:, 0:1], 1.0)
        g = jnp.maximum(_dot(pooled, l1w_ref[...]) + l1b_ref[...], 0.0)
        out_ref[...] = _dot(g, l2w_ref[...]) + l2b_ref[...]


def _row_spec(block_cols):
    return pl.BlockSpec((RB, block_cols), lambda i: (i, 0))


def _full_spec(shape):
    return pl.BlockSpec(shape, lambda i: tuple(0 for _ in shape))


_vec_spec = pl.BlockSpec((RB,), lambda i: (i,))
_both_halves = pl.BlockSpec((2, RB, D_H // 2), lambda i: (0, i, 0))
_out_split = pl.BlockSpec((2, RB, D_H // 2), lambda i: (0, i, 0))


def _tc1(x_pad, W1, deg_lo, deg_hi):
    return pl.pallas_call(
        _tc1_body,
        grid=(NRB,),
        in_specs=[_row_spec(D_IN), _full_spec((D_IN, D_H)), _vec_spec, _vec_spec],
        out_specs=_out_split,
        out_shape=jax.ShapeDtypeStruct((2, H, D_H // 2), FP),
    )(x_pad, W1, deg_lo, deg_hi)


def _tc_mid(agg, W, b, deg_lo, deg_hi):
    return pl.pallas_call(
        _tc_mid_body,
        grid=(NRB,),
        in_specs=[_both_halves, _full_spec((D_H, D_H)),
                  _full_spec((1, D_H)), _vec_spec, _vec_spec],
        out_specs=_out_split,
        out_shape=jax.ShapeDtypeStruct((2, H, D_H // 2), FP),
    )(agg, W, b, deg_lo, deg_hi)


def _tc_final(agg, b3, deg_lo, deg_hi, bidx, l1w, l1b, l2w_p, l2b_p):
    return pl.pallas_call(
        _tc_final_body,
        grid=(NRB,),
        in_specs=[_both_halves, _full_spec((1, D_H)), _vec_spec, _vec_spec,
                  _vec_spec, _full_spec((D_H, D_H)), _full_spec((1, D_H)),
                  _full_spec((D_H, 128)), _full_spec((1, 128))],
        out_specs=_full_spec((128, 128)),
        out_shape=jax.ShapeDtypeStruct((128, 128), FP),
        scratch_shapes=[pltpu.VMEM((128, D_H), FP), pltpu.VMEM((128, 8), FP)],
    )(agg, b3, deg_lo, deg_hi, bidx, l1w, l1b, l2w_p, l2b_p)


def kernel(x, edge_index, batch_idx, W1, b1, W2, b2, W3, b3,
           lin1_W, lin1_b, lin2_W, lin2_b):
    i32 = jnp.int32
    ei = edge_index.astype(i32)
    loop = jnp.arange(N, dtype=i32)
    npad = EPAD - ET
    padrow = N + (jnp.arange(npad, dtype=i32) % 128)
    src = jnp.concatenate([ei[0], loop, padrow])
    dst = jnp.concatenate([ei[1], loop, padrow])
    src2 = jnp.stack([src, src + H]).reshape(2, NS, NSUP, SW, K)
    dst_w = dst.reshape(NS, NSUP, SW, K)

    deg_flat = _deg_kernel(dst.reshape(NC, NS, NWIN_D, K))
    deg_lo, deg_hi = deg_flat[:H], deg_flat[H:]

    x_pad = jnp.pad(x.astype(FP), ((0, H - N), (0, 0)))
    b1r, b2r, b3r = b1[None, :], b2[None, :], b3[None, :]
    l1b = lin1_b[None, :]
    l2w_p = jnp.pad(lin2_W, ((0, 0), (0, 127)))
    l2b_p = jnp.pad(lin2_b, (0, 127))[None, :]
    bidx = jnp.concatenate(
        [batch_idx.astype(i32),
         G + (jnp.arange(H - N, dtype=i32) % (128 - G))])

    y = _tc1(x_pad, W1, deg_lo, deg_hi)
    agg = _agg_kernel(y.reshape(2 * H, D_H // 2), src2, dst_w).reshape(2, H, D_H // 2)
    y = _tc_mid(agg, W2, b1r, deg_lo, deg_hi)
    agg = _agg_kernel(y.reshape(2 * H, D_H // 2), src2, dst_w).reshape(2, H, D_H // 2)
    y = _tc_mid(agg, W3, b2r, deg_lo, deg_hi)
    agg = _agg_kernel(y.reshape(2 * H, D_H // 2), src2, dst_w).reshape(2, H, D_H // 2)
    out = _tc_final(agg, b3r, deg_lo, deg_hi, bidx, lin1_W, l1b, l2w_p, l2b_p)
    return out[:G, :1]
